# Initial kernel scaffold; baseline (speedup 1.0000x reference)
import jax
import jax.numpy as jnp
from jax.experimental import pallas as pl


def kernel(x, edge_index, batch, bn_feat_g, bn_feat_b, W_feat, b_feat, bn_g, bn_b, Ws, bs):
    raise NotImplementedError("write your pallas kernel here")


# speedup vs baseline: 14.7702x; 14.7702x over previous
import functools

import jax
import jax.numpy as jnp
from jax import lax
from jax.experimental import pallas as pl
from jax.experimental.pallas import tpu as pltpu
from jax.experimental.pallas import tpu_sc as plsc

N = 10000
D = 128
NUM_GRAPHS = 64
NUM_LAYERS = 3
EPS = 1e-5

NC = 2
NS = 16
LANES = 16
NW = NC * NS
BLK = 80
NPAD = 10240
RPT = NPAD // NS

_HIGH = jax.lax.Precision.HIGHEST
_DEF = jax.lax.Precision.DEFAULT


def _mesh():
    return plsc.VectorSubcoreMesh(core_axis_name="c", subcore_axis_name="s")


def _deg_counts(dst3, ones_rows, zerosD):
    nblk = dst3.shape[1]

    @functools.partial(
        pl.kernel, mesh=_mesh(),
        out_type=jax.ShapeDtypeStruct((NC, NPAD, D), jnp.float32),
        scratch_types=[
            pltpu.VMEM((nblk, BLK), jnp.int32),
            pltpu.VMEM((BLK, D), jnp.float32),
            pltpu.VMEM_SHARED((NPAD, D), jnp.float32),
        ],
    )
    def deg_kernel(dst_hbm, ones_hbm, z_hbm, out_hbm, dst_v, ones_v, acc_sh):
        cid = lax.axis_index("c")
        sid = lax.axis_index("s")
        wid = cid * NS + sid
        pltpu.sync_copy(z_hbm.at[pl.ds(sid * RPT, RPT)],
                        acc_sh.at[pl.ds(sid * RPT, RPT)])
        pltpu.sync_copy(ones_hbm, ones_v)
        pltpu.sync_copy(dst_hbm.at[wid], dst_v)
        plsc.subcore_barrier()

        @pl.loop(0, nblk)
        def _(i):
            pltpu.sync_copy(ones_v, acc_sh.at[dst_v.at[i]], add=True)

        plsc.subcore_barrier()
        pltpu.sync_copy(acc_sh.at[pl.ds(sid * RPT, RPT)],
                        out_hbm.at[cid, pl.ds(sid * RPT, RPT)])

    return deg_kernel(dst3, ones_rows, zerosD)


def _edge_aggregate(table, src3, dst3, zerosD):
    nblk = src3.shape[1]

    @functools.partial(
        pl.kernel, mesh=_mesh(),
        out_type=jax.ShapeDtypeStruct((NC, NPAD, D), jnp.float32),
        scratch_types=[
            pltpu.VMEM((nblk, BLK), jnp.int32),
            pltpu.VMEM((nblk, BLK), jnp.int32),
            pltpu.VMEM((BLK, D), jnp.float32),
            pltpu.VMEM_SHARED((NPAD, D), jnp.float32),
        ],
    )
    def agg_kernel(table_hbm, src_hbm, dst_hbm, z_hbm, out_hbm,
                   src_v, dst_v, rows_v, acc_sh):
        cid = lax.axis_index("c")
        sid = lax.axis_index("s")
        wid = cid * NS + sid
        pltpu.sync_copy(z_hbm.at[pl.ds(sid * RPT, RPT)],
                        acc_sh.at[pl.ds(sid * RPT, RPT)])
        pltpu.sync_copy(src_hbm.at[wid], src_v)
        pltpu.sync_copy(dst_hbm.at[wid], dst_v)
        plsc.subcore_barrier()

        @pl.loop(0, nblk)
        def _(i):
            pltpu.sync_copy(table_hbm.at[src_v.at[i]], rows_v)
            pltpu.sync_copy(rows_v, acc_sh.at[dst_v.at[i]], add=True)

        plsc.subcore_barrier()
        pltpu.sync_copy(acc_sh.at[pl.ds(sid * RPT, RPT)],
                        out_hbm.at[cid, pl.ds(sid * RPT, RPT)])

    return agg_kernel(table, src3, dst3, zerosD)


def _bn(x, g, b):
    mu = jnp.mean(x, axis=0, keepdims=True)
    var = jnp.mean((x - mu) ** 2, axis=0, keepdims=True)
    return (x - mu) / jnp.sqrt(var + EPS) * g + b


def _tc_prelude(x, g, b, W, bvec):
    def body(x_ref, g_ref, b_ref, w_ref, bv_ref, o_ref):
        hn = _bn(x_ref[...], g_ref[...], b_ref[...])
        o_ref[...] = jnp.maximum(
            jnp.dot(hn, w_ref[...], preferred_element_type=jnp.float32,
                    precision=_DEF) + bv_ref[...], 0.0)

    return pl.pallas_call(
        body, out_shape=jax.ShapeDtypeStruct((N, D), jnp.float32),
    )(x, g.reshape(1, D), b.reshape(1, D), W, bvec.reshape(1, D))


def _tc_layer0(h, deg_parts, g, b, W):
    def body(h_ref, dp_ref, g_ref, b_ref, w_ref, hs_ref, dinv_ref):
        deg = dp_ref[0, :N, :1] + dp_ref[1, :N, :1] + 1.0
        dinv = lax.rsqrt(jnp.maximum(deg, 1.0))
        dinv_ref[...] = dinv
        hn = _bn(h_ref[...], g_ref[...], b_ref[...])
        hs_ref[...] = jnp.dot(hn, w_ref[...], preferred_element_type=jnp.float32,
                              precision=_DEF) * dinv

    return pl.pallas_call(
        body, out_shape=(jax.ShapeDtypeStruct((N, D), jnp.float32),
                         jax.ShapeDtypeStruct((N, 1), jnp.float32)),
    )(h, deg_parts, g.reshape(1, D), b.reshape(1, D), W)


def _tc_mid(parts, hs_prev, dinv, bias, g, b, W):
    def body(p_ref, hsp_ref, dinv_ref, bias_ref, g_ref, b_ref, w_ref, hs_ref):
        dinv_v = dinv_ref[...]
        agg = dinv_v * (p_ref[0, :N, :] + p_ref[1, :N, :] + hsp_ref[...])
        h = jnp.maximum(agg + bias_ref[...], 0.0)
        hn = _bn(h, g_ref[...], b_ref[...])
        hs_ref[...] = jnp.dot(hn, w_ref[...], preferred_element_type=jnp.float32,
                              precision=_DEF) * dinv_v

    return pl.pallas_call(
        body, out_shape=jax.ShapeDtypeStruct((N, D), jnp.float32),
    )(parts, hs_prev, dinv, bias.reshape(1, D), g.reshape(1, D),
      b.reshape(1, D), W)


def _tc_final(parts, hs_prev, dinv, bias, batch_row):
    def body(p_ref, hsp_ref, dinv_ref, bias_ref, bt_ref, o_ref):
        dinv_v = dinv_ref[...]
        agg = dinv_v * (p_ref[0, :N, :] + p_ref[1, :N, :] + hsp_ref[...])
        h = jnp.maximum(agg + bias_ref[...], 0.0)
        gids = lax.broadcasted_iota(jnp.int32, (NUM_GRAPHS, N), 0)
        onehot = (gids == bt_ref[...]).astype(jnp.float32)
        o_ref[...] = jnp.dot(onehot, h, preferred_element_type=jnp.float32,
                             precision=_HIGH)

    return pl.pallas_call(
        body, out_shape=jax.ShapeDtypeStruct((NUM_GRAPHS, D), jnp.float32),
    )(parts, hs_prev, dinv, bias.reshape(1, D), batch_row)


def kernel(x, edge_index, batch, bn_feat_g, bn_feat_b, W_feat, b_feat,
           bn_g, bn_b, Ws, bs):
    E = edge_index.shape[1]
    src = edge_index[0].astype(jnp.int32)
    dst = edge_index[1].astype(jnp.int32)

    chunk = NW * BLK
    e_pad = ((E + chunk - 1) // chunk) * chunk
    if e_pad != E:
        src = jnp.concatenate([src, jnp.zeros((e_pad - E,), jnp.int32)])
        dst = jnp.concatenate(
            [dst, jnp.full((e_pad - E,), NPAD - 1, jnp.int32)])
    nblk = e_pad // chunk
    src3 = src.reshape(NW, nblk, BLK)
    dst3 = dst.reshape(NW, nblk, BLK)

    ones_rows = jnp.ones((BLK, D), jnp.float32)
    zerosD = jnp.zeros((NPAD, D), jnp.float32)

    deg_parts = _deg_counts(dst3, ones_rows, zerosD)
    h = _tc_prelude(x, bn_feat_g, bn_feat_b, W_feat, b_feat)

    hs, dinv = _tc_layer0(h, deg_parts, bn_g[0], bn_b[0], Ws[0])
    for i in range(NUM_LAYERS):
        parts = _edge_aggregate(hs, src3, dst3, zerosD)
        if i + 1 < NUM_LAYERS:
            hs = _tc_mid(parts, hs, dinv, bs[i], bn_g[i + 1], bn_b[i + 1],
                         Ws[i + 1])
        else:
            return _tc_final(parts, hs, dinv, bs[i],
                             batch.astype(jnp.int32).reshape(1, N))
